# baseline (speedup 1.0000x reference)
import functools

import jax
import jax.numpy as jnp
from jax.experimental import pallas as pl
from jax.experimental.pallas import tpu as pltpu

N = 4096
D = 256
K = 256
TR = 256
GRID = N // TR


def _scores_body(rf_ref, sf_ref, s_ref, r_ref, cacc_ref):
    t = pl.program_id(0)
    m = jax.lax.dot_general(
        rf_ref[...], sf_ref[...],
        dimension_numbers=(((1,), (1,)), ((), ())),
        preferred_element_type=jnp.float32,
    )
    s = jnp.exp(-(2.0 - 2.0 * m))
    s_ref[...] = s

    x = s.reshape(TR, 32, 128)
    acc = x[:, 0, :]
    for ci in range(1, 32):
        acc = acc + x[:, ci, :]
    w = acc.reshape(TR, 16, 8)
    v8 = w[:, 0, :]
    for k in range(1, 16):
        v8 = v8 + w[:, k, :]
    v4 = v8[:, :4] + v8[:, 4:]
    v2 = v4[:, :2] + v4[:, 2:]
    r = v2[:, 0] + v2[:, 1]
    r_ref[...] = r.reshape(1, 1, TR)

    @pl.when(t % 4 == 0)
    def _():
        cacc_ref[...] = jnp.zeros_like(cacc_ref)

    acc8 = cacc_ref[...].reshape(8, N)
    for rr in range(TR // 8):
        acc8 = acc8 + s[8 * rr:8 * rr + 8, :]
    cacc_ref[...] = acc8.reshape(1, 8, N)


def _scores_call(rf, sf):
    return pl.pallas_call(
        _scores_body,
        grid=(GRID,),
        in_specs=[
            pl.BlockSpec((TR, D), lambda i: (i, 0)),
            pl.BlockSpec((N, D), lambda i: (0, 0)),
        ],
        out_specs=[
            pl.BlockSpec((TR, N), lambda i: (i, 0)),
            pl.BlockSpec((1, 1, TR), lambda i: (i, 0, 0)),
            pl.BlockSpec((1, 8, N), lambda i: (i // 4, 0, 0)),
        ],
        out_shape=[
            jax.ShapeDtypeStruct((N, N), jnp.float32),
            jax.ShapeDtypeStruct((GRID, 1, TR), jnp.float32),
            jax.ShapeDtypeStruct((4, 8, N), jnp.float32),
        ],
    )(rf, sf)


def _normalize_body(s_ref, r_ref, cpart_ref, a_ref, cm_ref):
    cp = cpart_ref[...]
    f = []
    for q in range(4):
        a8 = cp[q]
        a4 = a8[:4] + a8[4:]
        a2 = a4[:2] + a4[2:]
        f.append(a2[0] + a2[1])
    c = ((f[0] + f[1]) + f[2]) + f[3]
    s = s_ref[...]
    r = r_ref[...].reshape(TR, 1)
    t1 = s / r
    t2 = s / c.reshape(1, N)
    a = t1 * t2
    a_ref[...] = a
    cm_ref[...] = jnp.max(a, axis=1).reshape(1, 1, TR)


def _normalize_call(s, r3, cpart):
    return pl.pallas_call(
        _normalize_body,
        grid=(GRID,),
        in_specs=[
            pl.BlockSpec((TR, N), lambda i: (i, 0)),
            pl.BlockSpec((1, 1, TR), lambda i: (i, 0, 0)),
            pl.BlockSpec((4, 8, N), lambda i: (0, 0, 0)),
        ],
        out_specs=[
            pl.BlockSpec((TR, N), lambda i: (i, 0)),
            pl.BlockSpec((1, 1, TR), lambda i: (i, 0, 0)),
        ],
        out_shape=[
            jax.ShapeDtypeStruct((N, N), jnp.float32),
            jax.ShapeDtypeStruct((GRID, 1, TR), jnp.float32),
        ],
    )(s, r3, cpart)


def _threshold_body(cm_ref, t_ref):
    xb = pltpu.bitcast(cm_ref[...], jnp.int32)
    t = jnp.int32(0)
    for bit in range(30, -1, -1):
        t2 = t | jnp.int32(1 << bit)
        cnt = jnp.sum((xb >= t2).astype(jnp.int32))
        t = jnp.where(cnt >= K, t2, t)
    t_ref[0, 0] = t


def _threshold_call(cm):
    return pl.pallas_call(
        _threshold_body,
        in_specs=[pl.BlockSpec((32, 128), lambda: (0, 0))],
        out_specs=pl.BlockSpec(memory_space=pltpu.SMEM),
        out_shape=jax.ShapeDtypeStruct((1, 1), jnp.int32),
    )(cm)


def kernel(ref_feats, src_feats, ref_masks, src_masks):
    s, r3, cpart = _scores_call(ref_feats, src_feats)
    a, cm3 = _normalize_call(s, r3, cpart)
    cm = cm3.reshape(32, 128)
    tbits = _threshold_call(cm)
    del tbits
    flat = a.reshape(-1)
    corr_scores, corr_indices = jax.lax.top_k(flat, K)
    ref_idx = corr_indices // N
    src_idx = corr_indices % N
    return (ref_idx, src_idx, corr_scores)
